# baseline (speedup 1.0000x reference)
import functools

import jax
import jax.numpy as jnp
from jax import lax
from jax.experimental import pallas as pl
from jax.experimental.pallas import tpu as pltpu
from jax.experimental.pallas import tpu_sc as plsc

NC = 2
NS = 16
NW = NC * NS
CHK = 128


def _edges_body(ei_ref, src_ref, dst_ref, *, n, npad, nchunks):
    src = ei_ref[0]
    dst = ei_ref[1]
    src_ref[:nchunks] = src
    dst_ref[:nchunks] = jnp.where(src == dst, n, dst)
    pad = src_ref.shape[0] - nchunks
    if pad:
        spare = npad - n - 1
        r = lax.broadcasted_iota(jnp.int32, (pad, CHK), 0)
        c = lax.broadcasted_iota(jnp.int32, (pad, CHK), 1)
        fill = n + 1 + lax.rem(r * CHK + c, jnp.int32(spare))
        src_ref[nchunks:] = fill
        dst_ref[nchunks:] = fill


def _dinv_from(degw, npad):
    deg = degw[0][:, :1] + degw[1][:, :1] + 1.0
    return lax.rsqrt(deg)


def _mm_body(x_ref, w_ref, b_ref, o_ref):
    o_ref[...] = jnp.dot(x_ref[...], w_ref[...],
                         preferred_element_type=jnp.float32) + b_ref[...]


def _scale_body(h_ref, degw_ref, o_ref, *, n, npad):
    dinv = _dinv_from(degw_ref, npad)
    rowid = lax.broadcasted_iota(jnp.int32, (npad, 1), 0)
    o_ref[...] = jnp.where(rowid < n, dinv * h_ref[...], 0.0)


def _l2_body(acc_ref, h1s_ref, degw_ref, g_ref, bt_ref, w_ref, b_ref, o_ref,
             *, n, npad):
    dinv = _dinv_from(degw_ref, npad)
    g = dinv * (acc_ref[0] + acc_ref[1] + h1s_ref[...])
    rowid = lax.broadcasted_iota(jnp.int32, (npad, 1), 0)
    rmask = (rowid < n).astype(jnp.float32)
    mean = jnp.sum(g, axis=0, keepdims=True) / n
    dev = (g - mean) * rmask
    var = jnp.sum(dev * dev, axis=0, keepdims=True) / n
    bn = g_ref[...] * (g - mean) * lax.rsqrt(var + 1e-5) + bt_ref[...]
    r = jnp.maximum(bn, 0.0)
    h2 = jnp.dot(r, w_ref[...], preferred_element_type=jnp.float32) + b_ref[...]
    o_ref[...] = jnp.where(rowid < n, dinv * h2, 0.0)


def _out_body(acc_ref, h2s_ref, degw_ref, o_ref, *, n, npad):
    dinv = _dinv_from(degw_ref, npad)
    o_ref[...] = (dinv * (acc_ref[0] + acc_ref[1] + h2s_ref[...]))[:n]


def _sc_edges_deg_body(ei_ref, zeros_ref, ones_ref,
                       deg_out, srcp_out, dstp_out,
                       deg_sh, sr_v, dr_v, so_v, do_v, ones_v, sem, semo,
                       *, n, npad, chpt, rpt, ept_raw):
    c = lax.axis_index("c")
    s = lax.axis_index("s")
    wid = s * NC + c
    rows = pl.ds(s * rpt, rpt)
    spare = npad - n - 1
    pltpu.sync_copy(zeros_ref.at[rows], deg_sh.at[rows])
    pltpu.sync_copy(ones_ref, ones_v)
    pltpu.sync_copy(ei_ref.at[0].at[pl.ds(wid * ept_raw, ept_raw)],
                    sr_v.at[pl.ds(0, ept_raw)])
    pltpu.sync_copy(ei_ref.at[1].at[pl.ds(wid * ept_raw, ept_raw)],
                    dr_v.at[pl.ds(0, ept_raw)])

    base = wid * chpt * CHK
    lane = lax.iota(jnp.int32, 16)

    def fix(j, carry):
        for g in range(CHK // 16):
            t = j * CHK + g * 16
            tv = t + lane
            real = tv < ept_raw
            sv = sr_v[pl.ds(t, 16)]
            dv = dr_v[pl.ds(t, 16)]
            padv = n + 1 + lax.rem(base + tv, jnp.int32(spare))
            so = jnp.where(real, sv, padv)
            do = jnp.where(real, jnp.where(sv == dv, jnp.int32(n), dv), padv)
            so_v[j, pl.ds(g * 16, 16)] = so
            do_v[j, pl.ds(g * 16, 16)] = do
        return carry

    lax.fori_loop(0, chpt, fix, 0)
    pltpu.async_copy(so_v, srcp_out.at[wid], semo)
    pltpu.async_copy(do_v, dstp_out.at[wid], semo)
    plsc.subcore_barrier()

    def body(g, carry):
        for u in range(NBUF):
            pltpu.async_copy(ones_v, deg_sh.at[do_v.at[NBUF * g + u]],
                             sem, add=True)
        for u in range(NBUF):
            pltpu.make_async_copy(ones_v, deg_sh.at[do_v.at[0]], sem).wait()
        return carry

    lax.fori_loop(0, chpt // NBUF, body, 0)
    pltpu.make_async_copy(so_v, srcp_out.at[wid], semo).wait()
    pltpu.make_async_copy(do_v, dstp_out.at[wid], semo).wait()
    plsc.subcore_barrier()
    pltpu.sync_copy(deg_sh.at[rows], deg_out.at[c].at[rows])


NBUF = 3


def _sc_spmm_body(srcp_ref, dstp_ref, hs_ref, zeros_ref, out_ref,
                  acc_sh, sring, dring, rows_v,
                  gs0, gs1, gs2, ds0, ds1, ds2, ss0, ss1, ss2,
                  *, chpt, rpt):
    c = lax.axis_index("c")
    s = lax.axis_index("s")
    wid = s * NC + c
    gsem = (gs0, gs1, gs2)
    dsem = (ds0, ds1, ds2)
    ssem = (ss0, ss1, ss2)
    rows = pl.ds(s * rpt, rpt)
    src_t = srcp_ref.at[wid]
    dst_t = dstp_ref.at[wid]
    pltpu.sync_copy(zeros_ref.at[rows], acc_sh.at[rows])
    for u in range(NBUF):
        pltpu.async_copy(src_t.at[u], sring.at[u], ssem[u])
        pltpu.async_copy(dst_t.at[u], dring.at[u], dsem[u])
    plsc.subcore_barrier()
    for u in range(2):
        pltpu.make_async_copy(dst_t.at[u], dring.at[u], dsem[u]).wait()
        pltpu.async_copy(hs_ref.at[dring.at[u]], rows_v.at[u], gsem[u])

    def body(i, carry):
        j0 = 3 * i
        for u in range(NBUF):
            j = j0 + u
            b = u
            b2 = (u + 2) % NBUF
            pltpu.make_async_copy(hs_ref.at[dring.at[b]], rows_v.at[b],
                                  gsem[b]).wait()

            @pl.when(j + 2 < chpt)
            def _(b2=b2, j=j):
                pltpu.make_async_copy(dst_t.at[j + 2], dring.at[b2],
                                      dsem[b2]).wait()
                pltpu.async_copy(hs_ref.at[dring.at[b2]], rows_v.at[b2],
                                 gsem[b2])

            pltpu.make_async_copy(src_t.at[j], sring.at[b], ssem[b]).wait()
            pltpu.sync_copy(rows_v.at[b], acc_sh.at[sring.at[b]], add=True)

            @pl.when(j + 3 < chpt)
            def _(b=b, j=j):
                pltpu.async_copy(src_t.at[j + 3], sring.at[b], ssem[b])
                pltpu.async_copy(dst_t.at[j + 3], dring.at[b], dsem[b])
        return carry

    lax.fori_loop(0, chpt // NBUF, body, 0)
    plsc.subcore_barrier()
    pltpu.sync_copy(acc_sh.at[rows], out_ref.at[c].at[rows])


def kernel(x, edge_index, W1, b1, gamma1, beta1, W2, b2):
    n, d = x.shape
    e = edge_index.shape[1]
    nchunks = e // CHK
    assert nchunks * CHK == e
    chpt = -(-nchunks // NW)
    chpt = ((chpt + NBUF - 1) // NBUF) * NBUF
    npad = ((n + 1 + 127) // 128) * 128
    rpt = npad // NS

    mesh = plsc.VectorSubcoreMesh(core_axis_name="c", subcore_axis_name="s")

    zeros16 = jnp.zeros((npad, 16), jnp.float32)
    zeros128 = jnp.zeros((npad, d), jnp.float32)
    ones16 = jnp.ones((CHK, 16), jnp.float32)
    x_pad = jnp.pad(x, ((0, npad - n), (0, 0)))

    assert e % NW == 0
    deg_w, srcp, dstp = pl.kernel(
        functools.partial(_sc_edges_deg_body, n=n, npad=npad, chpt=chpt,
                          rpt=rpt, ept_raw=e // NW),
        out_type=[
            jax.ShapeDtypeStruct((NC, npad, 16), jnp.float32),
            jax.ShapeDtypeStruct((NW, chpt, CHK), jnp.int32),
            jax.ShapeDtypeStruct((NW, chpt, CHK), jnp.int32),
        ],
        mesh=mesh,
        compiler_params=pltpu.CompilerParams(use_tc_tiling_on_sc=False),
        scratch_types=[
            pltpu.VMEM_SHARED((npad, 16), jnp.float32),
            pltpu.VMEM((chpt * CHK,), jnp.int32),
            pltpu.VMEM((chpt * CHK,), jnp.int32),
            pltpu.VMEM((chpt, CHK), jnp.int32),
            pltpu.VMEM((chpt, CHK), jnp.int32),
            pltpu.VMEM((CHK, 16), jnp.float32),
            pltpu.SemaphoreType.DMA,
            pltpu.SemaphoreType.DMA,
        ],
    )(edge_index, zeros16, ones16)

    h1 = pl.pallas_call(
        _mm_body,
        out_shape=jax.ShapeDtypeStruct((npad, d), jnp.float32),
    )(x_pad, W1, b1.reshape(1, d))
    h1s = pl.pallas_call(
        functools.partial(_scale_body, n=n, npad=npad),
        out_shape=jax.ShapeDtypeStruct((npad, d), jnp.float32),
    )(h1, deg_w)

    spmm = pl.kernel(
        functools.partial(_sc_spmm_body, chpt=chpt, rpt=rpt),
        out_type=jax.ShapeDtypeStruct((NC, npad, d), jnp.float32),
        mesh=mesh,
        scratch_types=[
            pltpu.VMEM_SHARED((npad, d), jnp.float32),
            pltpu.VMEM((NBUF, CHK), jnp.int32),
            pltpu.VMEM((NBUF, CHK), jnp.int32),
            pltpu.VMEM((NBUF, CHK, d), jnp.float32),
        ] + [pltpu.SemaphoreType.DMA] * 9,
    )

    acc1 = spmm(srcp, dstp, h1s, zeros128)

    h2s = pl.pallas_call(
        functools.partial(_l2_body, n=n, npad=npad),
        out_shape=jax.ShapeDtypeStruct((npad, d), jnp.float32),
    )(acc1, h1s, deg_w, gamma1.reshape(1, d), beta1.reshape(1, d),
      W2, b2.reshape(1, d))

    acc2 = spmm(srcp, dstp, h2s, zeros128)

    out = pl.pallas_call(
        functools.partial(_out_body, n=n, npad=npad),
        out_shape=jax.ShapeDtypeStruct((n, d), jnp.float32),
    )(acc2, h2s, deg_w)
    return out


# speedup vs baseline: 1.0320x; 1.0320x over previous
import functools

import jax
import jax.numpy as jnp
from jax import lax
from jax.experimental import pallas as pl
from jax.experimental.pallas import tpu as pltpu
from jax.experimental.pallas import tpu_sc as plsc

NC = 2
NS = 16
NW = NC * NS
CHK = 128


def _edges_body(ei_ref, src_ref, dst_ref, *, n, npad, nchunks):
    src = ei_ref[0]
    dst = ei_ref[1]
    src_ref[:nchunks] = src
    dst_ref[:nchunks] = jnp.where(src == dst, n, dst)
    pad = src_ref.shape[0] - nchunks
    if pad:
        spare = npad - n - 1
        r = lax.broadcasted_iota(jnp.int32, (pad, CHK), 0)
        c = lax.broadcasted_iota(jnp.int32, (pad, CHK), 1)
        fill = n + 1 + lax.rem(r * CHK + c, jnp.int32(spare))
        src_ref[nchunks:] = fill
        dst_ref[nchunks:] = fill


def _dinv_from(degw, npad):
    deg = degw[0][:, :1] + degw[1][:, :1] + 1.0
    return lax.rsqrt(deg)


def _m1_body(x_ref, w_ref, b_ref, degw_ref, o_ref, *, n, npad):
    dinv = _dinv_from(degw_ref, npad)
    h = jnp.dot(x_ref[...], w_ref[...], preferred_element_type=jnp.float32)
    rowid = lax.broadcasted_iota(jnp.int32, (npad, 1), 0)
    o_ref[...] = jnp.where(rowid < n, dinv * (h + b_ref[...]), 0.0)


def _l2_body(acc_ref, h1s_ref, degw_ref, g_ref, bt_ref, w_ref, b_ref, o_ref,
             *, n, npad):
    dinv = _dinv_from(degw_ref, npad)
    g = dinv * (acc_ref[0] + acc_ref[1] + h1s_ref[...])
    rowid = lax.broadcasted_iota(jnp.int32, (npad, 1), 0)
    rmask = (rowid < n).astype(jnp.float32)
    mean = jnp.sum(g, axis=0, keepdims=True) / n
    dev = (g - mean) * rmask
    var = jnp.sum(dev * dev, axis=0, keepdims=True) / n
    bn = g_ref[...] * (g - mean) * lax.rsqrt(var + 1e-5) + bt_ref[...]
    r = jnp.maximum(bn, 0.0)
    h2 = jnp.dot(r, w_ref[...], preferred_element_type=jnp.float32) + b_ref[...]
    o_ref[...] = jnp.where(rowid < n, dinv * h2, 0.0)


def _out_body(acc_ref, h2s_ref, degw_ref, o_ref, *, n, npad):
    dinv = _dinv_from(degw_ref, npad)
    o_ref[...] = (dinv * (acc_ref[0] + acc_ref[1] + h2s_ref[...]))[:n]


def _sc_deg_body(dstp_ref, zeros_ref, ones_ref, out_ref,
                 deg_sh, idx_v, ones_v, sem, *, chpt, rpt):
    c = lax.axis_index("c")
    s = lax.axis_index("s")
    wid = s * NC + c
    rows = pl.ds(s * rpt, rpt)
    pltpu.sync_copy(zeros_ref.at[rows], deg_sh.at[rows])
    pltpu.sync_copy(ones_ref, ones_v)
    pltpu.sync_copy(dstp_ref.at[wid], idx_v)
    plsc.subcore_barrier()

    def body(g, carry):
        for u in range(NBUF):
            pltpu.async_copy(ones_v, deg_sh.at[idx_v.at[NBUF * g + u]],
                             sem, add=True)
        for u in range(NBUF):
            pltpu.make_async_copy(ones_v, deg_sh.at[idx_v.at[0]], sem).wait()
        return carry

    lax.fori_loop(0, chpt // NBUF, body, 0)
    plsc.subcore_barrier()
    pltpu.sync_copy(deg_sh.at[rows], out_ref.at[c].at[rows])


NBUF = 3


def _sc_spmm_body(srcp_ref, dstp_ref, hs_ref, zeros_ref, out_ref,
                  acc_sh, sring, dring, rows_v,
                  gs0, gs1, gs2, ds0, ds1, ds2, ss0, ss1, ss2,
                  *, chpt, rpt):
    c = lax.axis_index("c")
    s = lax.axis_index("s")
    wid = s * NC + c
    gsem = (gs0, gs1, gs2)
    dsem = (ds0, ds1, ds2)
    ssem = (ss0, ss1, ss2)
    rows = pl.ds(s * rpt, rpt)
    src_t = srcp_ref.at[wid]
    dst_t = dstp_ref.at[wid]
    pltpu.sync_copy(zeros_ref.at[rows], acc_sh.at[rows])
    for u in range(NBUF):
        pltpu.async_copy(src_t.at[u], sring.at[u], ssem[u])
        pltpu.async_copy(dst_t.at[u], dring.at[u], dsem[u])
    plsc.subcore_barrier()
    for u in range(2):
        pltpu.make_async_copy(dst_t.at[u], dring.at[u], dsem[u]).wait()
        pltpu.async_copy(hs_ref.at[dring.at[u]], rows_v.at[u], gsem[u])

    def body(i, carry):
        j0 = 3 * i
        for u in range(NBUF):
            j = j0 + u
            b = u
            b2 = (u + 2) % NBUF
            pltpu.make_async_copy(hs_ref.at[dring.at[b]], rows_v.at[b],
                                  gsem[b]).wait()

            @pl.when(j + 2 < chpt)
            def _(b2=b2, j=j):
                pltpu.make_async_copy(dst_t.at[j + 2], dring.at[b2],
                                      dsem[b2]).wait()
                pltpu.async_copy(hs_ref.at[dring.at[b2]], rows_v.at[b2],
                                 gsem[b2])

            pltpu.make_async_copy(src_t.at[j], sring.at[b], ssem[b]).wait()
            pltpu.sync_copy(rows_v.at[b], acc_sh.at[sring.at[b]], add=True)

            @pl.when(j + 3 < chpt)
            def _(b=b, j=j):
                pltpu.async_copy(src_t.at[j + 3], sring.at[b], ssem[b])
                pltpu.async_copy(dst_t.at[j + 3], dring.at[b], dsem[b])
        return carry

    lax.fori_loop(0, chpt // NBUF, body, 0)
    plsc.subcore_barrier()
    pltpu.sync_copy(acc_sh.at[rows], out_ref.at[c].at[rows])


def kernel(x, edge_index, W1, b1, gamma1, beta1, W2, b2):
    n, d = x.shape
    e = edge_index.shape[1]
    nchunks = e // CHK
    assert nchunks * CHK == e
    chpt = -(-nchunks // NW)
    chpt = ((chpt + NBUF - 1) // NBUF) * NBUF
    npad = ((n + 1 + 127) // 128) * 128
    rpt = npad // NS

    mesh = plsc.VectorSubcoreMesh(core_axis_name="c", subcore_axis_name="s")

    ei3 = edge_index.reshape(2, nchunks, CHK)
    srcp, dstp = pl.pallas_call(
        functools.partial(_edges_body, n=n, npad=npad, nchunks=nchunks),
        out_shape=[jax.ShapeDtypeStruct((NW * chpt, CHK), jnp.int32)] * 2,
    )(ei3)
    srcp = srcp.reshape(NW, chpt, CHK)
    dstp = dstp.reshape(NW, chpt, CHK)

    zeros16 = jnp.zeros((npad, 16), jnp.float32)
    zeros128 = jnp.zeros((npad, d), jnp.float32)
    ones16 = jnp.ones((CHK, 16), jnp.float32)
    x_pad = jnp.pad(x, ((0, npad - n), (0, 0)))

    deg_w = pl.kernel(
        functools.partial(_sc_deg_body, chpt=chpt, rpt=rpt),
        out_type=jax.ShapeDtypeStruct((NC, npad, 16), jnp.float32),
        mesh=mesh,
        compiler_params=pltpu.CompilerParams(use_tc_tiling_on_sc=False),
        scratch_types=[
            pltpu.VMEM_SHARED((npad, 16), jnp.float32),
            pltpu.VMEM((chpt, CHK), jnp.int32),
            pltpu.VMEM((CHK, 16), jnp.float32),
            pltpu.SemaphoreType.DMA,
        ],
    )(dstp, zeros16, ones16)

    h1s = pl.pallas_call(
        functools.partial(_m1_body, n=n, npad=npad),
        out_shape=jax.ShapeDtypeStruct((npad, d), jnp.float32),
    )(x_pad, W1, b1.reshape(1, d), deg_w)

    spmm = pl.kernel(
        functools.partial(_sc_spmm_body, chpt=chpt, rpt=rpt),
        out_type=jax.ShapeDtypeStruct((NC, npad, d), jnp.float32),
        mesh=mesh,
        scratch_types=[
            pltpu.VMEM_SHARED((npad, d), jnp.float32),
            pltpu.VMEM((NBUF, CHK), jnp.int32),
            pltpu.VMEM((NBUF, CHK), jnp.int32),
            pltpu.VMEM((NBUF, CHK, d), jnp.float32),
        ] + [pltpu.SemaphoreType.DMA] * 9,
    )

    acc1 = spmm(srcp, dstp, h1s, zeros128)

    h2s = pl.pallas_call(
        functools.partial(_l2_body, n=n, npad=npad),
        out_shape=jax.ShapeDtypeStruct((npad, d), jnp.float32),
    )(acc1, h1s, deg_w, gamma1.reshape(1, d), beta1.reshape(1, d),
      W2, b2.reshape(1, d))

    acc2 = spmm(srcp, dstp, h2s, zeros128)

    out = pl.pallas_call(
        functools.partial(_out_body, n=n, npad=npad),
        out_shape=jax.ShapeDtypeStruct((n, d), jnp.float32),
    )(acc2, h2s, deg_w)
    return out
